# baseline (speedup 1.0000x reference)
import functools

import jax
import jax.numpy as jnp
from jax import lax
from jax.experimental import pallas as pl
from jax.experimental.pallas import tpu as pltpu
from jax.experimental.pallas import tpu_sc as plsc

N_NODES = 10000
N_EDGES = 320000
D_FEAT = 128

NC = 2
NS = 16
L = 16
NW = NC * NS
EDGES_PER_TILE = N_EDGES // NW
BLK = 40
NBUF = 5
NBLK = EDGES_PER_TILE // BLK
NSUP = NBLK // NBUF
ROW_CHUNK = 624
ROW_TAIL = N_NODES - NS * ROW_CHUNK

_mesh = plsc.VectorSubcoreMesh(core_axis_name="c", subcore_axis_name="s")


@functools.partial(
    pl.kernel,
    out_type=jax.ShapeDtypeStruct((NC, N_NODES, D_FEAT), jnp.float32),
    mesh=_mesh,
    scratch_types=[
        pltpu.VMEM((EDGES_PER_TILE,), jnp.int32),
        pltpu.VMEM((EDGES_PER_TILE,), jnp.float32),
        [pltpu.VMEM((BLK,), jnp.int32) for _ in range(NBUF)],
        [pltpu.VMEM((BLK, D_FEAT), jnp.float32) for _ in range(NBUF)],
        pltpu.VMEM_SHARED((N_NODES, D_FEAT), jnp.float32),
        [pltpu.SemaphoreType.DMA for _ in range(NBUF)],
        [pltpu.SemaphoreType.DMA for _ in range(NBUF)],
        [pltpu.SemaphoreType.DMA for _ in range(NBUF)],
    ],
    compiler_params=pltpu.CompilerParams(needs_layout_passes=False),
)
def _sc_aggregate(node_hbm, zeros_hbm, src_hbm, dst_hbm, attr_hbm, part_hbm,
                  srcv, attrv, dstv, rows, accum, gsem, dsem, ssem):
    c = lax.axis_index("c")
    s = lax.axis_index("s")
    wid = c * NS + s

    rsl = pl.ds(s * ROW_CHUNK, ROW_CHUNK)
    tsl = pl.ds(NS * ROW_CHUNK, ROW_TAIL)

    @pl.when(c == 0)
    def _():
        pltpu.sync_copy(node_hbm.at[rsl], accum.at[rsl])

        @pl.when(s == 0)
        def _():
            pltpu.sync_copy(node_hbm.at[tsl], accum.at[tsl])

    @pl.when(c != 0)
    def _():
        pltpu.sync_copy(zeros_hbm.at[rsl], accum.at[rsl])

        @pl.when(s == 0)
        def _():
            pltpu.sync_copy(zeros_hbm.at[tsl], accum.at[tsl])

    ebase = wid * EDGES_PER_TILE
    pltpu.sync_copy(src_hbm.at[pl.ds(ebase, EDGES_PER_TILE)], srcv)
    pltpu.sync_copy(attr_hbm.at[pl.ds(ebase, EDGES_PER_TILE)], attrv)

    plsc.subcore_barrier()

    def start_block(j, k):
        off = j * BLK
        pltpu.async_copy(dst_hbm.at[pl.ds(ebase + off, BLK)], dstv[k], dsem[k])
        pltpu.async_copy(node_hbm.at[srcv.at[pl.ds(off, BLK)]], rows[k],
                         gsem[k])

    def finish_block(j, k):
        off = j * BLK
        pltpu.make_async_copy(node_hbm.at[srcv.at[pl.ds(off, BLK)]], rows[k],
                              gsem[k]).wait()

        @plsc.parallel_loop(0, BLK, unroll=16)
        def _scale(e):
            bc = plsc.load_gather(
                attrv, [jnp.full((L,), off, jnp.int32) + e])
            for cc in range(D_FEAT // L):
                sl = pl.ds(cc * L, L)
                rows[k][e, sl] = rows[k][e, sl] * bc

        pltpu.make_async_copy(dst_hbm.at[pl.ds(ebase + off, BLK)], dstv[k],
                              dsem[k]).wait()
        pltpu.async_copy(rows[k], accum.at[dstv[k]], ssem[k], add=True)

    def drain_scatter(k):
        pltpu.make_async_copy(rows[k], accum.at[dstv[k]], ssem[k]).wait()

    for k in range(NBUF):
        start_block(k, k)

    @pl.loop(0, NSUP - 1)
    def _super(t):
        j0 = t * NBUF
        for k in range(NBUF):
            finish_block(j0 + k, k)
        for k in range(NBUF):
            drain_scatter(k)
            start_block(j0 + NBUF + k, k)

    for k in range(NBUF):
        finish_block((NSUP - 1) * NBUF + k, k)
    for k in range(NBUF):
        drain_scatter(k)

    plsc.subcore_barrier()
    pltpu.sync_copy(accum.at[rsl], part_hbm.at[c, rsl])

    @pl.when(s == 0)
    def _():
        pltpu.sync_copy(accum.at[tsl], part_hbm.at[c, tsl])


def _combine_body(p_ref, o_ref):
    o_ref[...] = p_ref[0] + p_ref[1]


_combine = pl.pallas_call(
    _combine_body,
    out_shape=jax.ShapeDtypeStruct((N_NODES, D_FEAT), jnp.float32),
    grid=(10,),
    in_specs=[pl.BlockSpec((NC, N_NODES // 10, D_FEAT), lambda i: (0, i, 0))],
    out_specs=pl.BlockSpec((N_NODES // 10, D_FEAT), lambda i: (i, 0)),
)


@jax.jit
def kernel(node_feat, edge_index, edge_attr):
    src = edge_index[0].astype(jnp.int32)
    dst = edge_index[1].astype(jnp.int32)
    zeros = jnp.zeros_like(node_feat)
    part = _sc_aggregate(node_feat, zeros, src, dst, edge_attr)
    return _combine(part)


# speedup vs baseline: 1.0765x; 1.0765x over previous
import functools

import jax
import jax.numpy as jnp
from jax import lax
from jax.experimental import pallas as pl
from jax.experimental.pallas import tpu as pltpu
from jax.experimental.pallas import tpu_sc as plsc

N_NODES = 10000
N_EDGES = 320000
D_FEAT = 128

NC = 2
NS = 16
L = 16
NW = NC * NS
EDGES_PER_TILE = N_EDGES // NW
BLK = 80
NBUF = 3
NBLK = EDGES_PER_TILE // BLK
NFULL = 41
TAIL = NBLK - NFULL * NBUF
ROW_CHUNK = 624
ROW_TAIL = N_NODES - NS * ROW_CHUNK

_mesh = plsc.VectorSubcoreMesh(core_axis_name="c", subcore_axis_name="s")


@functools.partial(
    pl.kernel,
    out_type=jax.ShapeDtypeStruct((NC, N_NODES, D_FEAT), jnp.float32),
    mesh=_mesh,
    scratch_types=[
        pltpu.VMEM((EDGES_PER_TILE,), jnp.int32),
        [pltpu.VMEM((BLK,), jnp.float32) for _ in range(NBUF)],
        [pltpu.VMEM((BLK,), jnp.int32) for _ in range(NBUF)],
        [pltpu.VMEM((BLK, D_FEAT), jnp.float32) for _ in range(NBUF)],
        pltpu.VMEM_SHARED((N_NODES, D_FEAT), jnp.float32),
        [pltpu.SemaphoreType.DMA for _ in range(NBUF)],
        [pltpu.SemaphoreType.DMA for _ in range(NBUF)],
        [pltpu.SemaphoreType.DMA for _ in range(NBUF)],
        [pltpu.SemaphoreType.DMA for _ in range(NBUF)],
    ],
    compiler_params=pltpu.CompilerParams(needs_layout_passes=False),
)
def _sc_aggregate(node_hbm, zeros_hbm, src_hbm, dst_hbm, attr_hbm, part_hbm,
                  srcv, attrv, dstv, rows, accum, gsem, dsem, asem, ssem):
    c = lax.axis_index("c")
    s = lax.axis_index("s")
    wid = c * NS + s

    rsl = pl.ds(s * ROW_CHUNK, ROW_CHUNK)
    tsl = pl.ds(NS * ROW_CHUNK, ROW_TAIL)

    @pl.when(c == 0)
    def _():
        pltpu.sync_copy(node_hbm.at[rsl], accum.at[rsl])

        @pl.when(s == 0)
        def _():
            pltpu.sync_copy(node_hbm.at[tsl], accum.at[tsl])

    @pl.when(c != 0)
    def _():
        pltpu.sync_copy(zeros_hbm.at[rsl], accum.at[rsl])

        @pl.when(s == 0)
        def _():
            pltpu.sync_copy(zeros_hbm.at[tsl], accum.at[tsl])

    ebase = wid * EDGES_PER_TILE
    pltpu.sync_copy(src_hbm.at[pl.ds(ebase, EDGES_PER_TILE)], srcv)

    plsc.subcore_barrier()

    def start_block(j, k):
        off = j * BLK
        pltpu.async_copy(dst_hbm.at[pl.ds(ebase + off, BLK)], dstv[k], dsem[k])
        pltpu.async_copy(attr_hbm.at[pl.ds(ebase + off, BLK)], attrv[k],
                         asem[k])
        pltpu.async_copy(node_hbm.at[srcv.at[pl.ds(off, BLK)]], rows[k],
                         gsem[k])

    def finish_block(j, k):
        off = j * BLK
        pltpu.make_async_copy(node_hbm.at[srcv.at[pl.ds(off, BLK)]], rows[k],
                              gsem[k]).wait()
        pltpu.make_async_copy(attr_hbm.at[pl.ds(ebase + off, BLK)], attrv[k],
                              asem[k]).wait()

        @plsc.parallel_loop(0, BLK, unroll=8)
        def _scale(e):
            bc = plsc.load_gather(attrv[k], [jnp.full((L,), 0, jnp.int32) + e])
            for cc in range(D_FEAT // L):
                sl = pl.ds(cc * L, L)
                rows[k][e, sl] = rows[k][e, sl] * bc

        pltpu.make_async_copy(dst_hbm.at[pl.ds(ebase + off, BLK)], dstv[k],
                              dsem[k]).wait()
        pltpu.async_copy(rows[k], accum.at[dstv[k]], ssem[k], add=True)

    def drain_scatter(k):
        pltpu.make_async_copy(rows[k], accum.at[dstv[k]], ssem[k]).wait()

    for k in range(NBUF):
        start_block(k, k)

    @pl.loop(0, NFULL - 1)
    def _super(t):
        j0 = t * NBUF
        for k in range(NBUF):
            finish_block(j0 + k, k)
        for k in range(NBUF):
            drain_scatter(k)
            start_block(j0 + NBUF + k, k)

    j0 = (NFULL - 1) * NBUF
    for k in range(NBUF):
        finish_block(j0 + k, k)
    for k in range(NBUF):
        drain_scatter(k)
        if k < TAIL:
            start_block(j0 + NBUF + k, k)
    for k in range(TAIL):
        finish_block(NFULL * NBUF + k, k)
    for k in range(TAIL):
        drain_scatter(k)

    plsc.subcore_barrier()
    pltpu.sync_copy(accum.at[rsl], part_hbm.at[c, rsl])

    @pl.when(s == 0)
    def _():
        pltpu.sync_copy(accum.at[tsl], part_hbm.at[c, tsl])


def _combine_body(p_ref, o_ref):
    o_ref[...] = p_ref[0] + p_ref[1]


_combine = pl.pallas_call(
    _combine_body,
    out_shape=jax.ShapeDtypeStruct((N_NODES, D_FEAT), jnp.float32),
    grid=(10,),
    in_specs=[pl.BlockSpec((NC, N_NODES // 10, D_FEAT), lambda i: (0, i, 0))],
    out_specs=pl.BlockSpec((N_NODES // 10, D_FEAT), lambda i: (i, 0)),
)


@jax.jit
def kernel(node_feat, edge_index, edge_attr):
    src = edge_index[0].astype(jnp.int32)
    dst = edge_index[1].astype(jnp.int32)
    zeros = jnp.zeros_like(node_feat)
    part = _sc_aggregate(node_feat, zeros, src, dst, edge_attr)
    return _combine(part)
